# baseline (speedup 1.0000x reference)
import functools

import jax
import jax.numpy as jnp
from jax import lax
from jax.experimental import pallas as pl
from jax.experimental.pallas import tpu as pltpu
from jax.experimental.pallas import tpu_sc as plsc

N = 10000
E = 320000
D = 128
T = 2

NC = 2
NS = 16
LANES = 128
G = 32
KSEG = 20
NBUF = 8
SEG_C0 = 30
SEG_C1 = 2
TOT_SEG = NS * (SEG_C0 + SEG_C1)
EPT = 16 * KSEG * G
NVEC = EPT // 16
E_PAD = TOT_SEG * KSEG * G
N_PAD = 10240
ROWS_PER_SUB = N_PAD // NS
HR = N_PAD // 128
BN = 2000
GRID = N // BN


def _deg_body(dst_hbm, zeros_hbm, iota_hbm, deg_out,
              idx_v, hist, col_v, i80_v, acc):
    cid = lax.axis_index("c")
    sid = lax.axis_index("s")
    pltpu.sync_copy(zeros_hbm, hist)

    @pl.when(sid == 0)
    def _():
        pltpu.sync_copy(hist, acc)

    plsc.subcore_barrier()
    pltpu.sync_copy(dst_hbm.at[cid, sid], idx_v)
    pltpu.sync_copy(iota_hbm, i80_v)

    @pl.loop(0, NVEC)
    def _(j):
        d = idx_v[j]
        cnt, last = plsc.scan_count(d)
        r = lax.shift_right_logical(d, 7)
        c = lax.bitwise_and(d, 127)
        plsc.addupdate_scatter(hist, [r, c], cnt.astype(jnp.float32),
                               mask=last)

    pltpu.sync_copy(hist, acc.at[i80_v], add=True)
    plsc.subcore_barrier()
    pltpu.sync_copy(acc.at[pl.ds(sid * 5, 5)], hist.at[pl.ds(0, 5)])
    for kk in range(ROWS_PER_SUB // 16):
        col_v[pl.ds(kk * 16, 16)] = hist[kk // 8, pl.ds((kk % 8) * 16, 16)]
    pltpu.sync_copy(col_v, deg_out.at[cid, pl.ds(sid * ROWS_PER_SUB,
                                                 ROWS_PER_SUB)])


def _agg_body(y_hbm, eidx_hbm, zeros_hbm, agg_out,
              idx_v, bufs, gsems, ssems, acc):
    cid = lax.axis_index("c")
    sid = lax.axis_index("s")

    pltpu.sync_copy(zeros_hbm, bufs[0])
    for t in range(ROWS_PER_SUB // G):
        pltpu.sync_copy(
            bufs[0], acc.at[pl.ds(sid * ROWS_PER_SUB + t * G, G)])
    plsc.subcore_barrier()

    base = jnp.where(cid == 0, sid * SEG_C0, NS * SEG_C0 + sid * SEG_C1)
    nseg = jnp.where(cid == 0, SEG_C0, SEG_C1)

    @pl.loop(0, nseg)
    def _(seg):
        pltpu.sync_copy(eidx_hbm.at[base + seg], idx_v)
        hg = [None] * NBUF
        hs = [None] * NBUF

        def gather(j):
            b = j % NBUF
            if hs[b] is not None:
                hs[b].wait()
            hg[b] = pltpu.async_copy(
                y_hbm.at[idx_v.at[0, j]], bufs[b], gsems[b])

        def scat(j):
            b = j % NBUF
            hg[b].wait()
            hs[b] = pltpu.async_copy(
                bufs[b], acc.at[idx_v.at[1, j]], ssems[b], add=True)

        for j in range(KSEG):
            gather(j)
            jj = j - (NBUF - 1)
            if jj >= 0:
                scat(jj)
        for jj in range(KSEG - NBUF + 1, KSEG):
            scat(jj)
        for b in range(NBUF):
            hs[b].wait()

    plsc.subcore_barrier()
    for t in range(ROWS_PER_SUB // G):
        base = sid * ROWS_PER_SUB + t * G
        pltpu.sync_copy(acc.at[pl.ds(base, G)], bufs[0])
        pltpu.sync_copy(bufs[0], agg_out.at[cid, pl.ds(base, G)])


@functools.cache
def _sc_kernels():
    mesh = plsc.VectorSubcoreMesh(core_axis_name="c", subcore_axis_name="s")
    deg_kernel = pl.kernel(
        _deg_body,
        out_type=jax.ShapeDtypeStruct((NC, N_PAD), jnp.float32),
        mesh=mesh,
        compiler_params=pltpu.CompilerParams(needs_layout_passes=False),
        scratch_types=[
            pltpu.VMEM((NVEC, 16), jnp.int32),
            pltpu.VMEM((HR, 128), jnp.float32),
            pltpu.VMEM((ROWS_PER_SUB,), jnp.float32),
            pltpu.VMEM((HR,), jnp.int32),
            pltpu.VMEM_SHARED((HR, 128), jnp.float32),
        ],
    )
    agg_kernel = pl.kernel(
        _agg_body,
        out_type=jax.ShapeDtypeStruct((NC, N_PAD, D), jnp.float32),
        mesh=mesh,
        scratch_types=[
            pltpu.VMEM((2, KSEG, G), jnp.int32),
            [pltpu.VMEM((G, D), jnp.float32) for _ in range(NBUF)],
            [pltpu.SemaphoreType.DMA for _ in range(NBUF)],
            [pltpu.SemaphoreType.DMA for _ in range(NBUF)],
            pltpu.VMEM_SHARED((N_PAD, D), jnp.float32),
        ],
    )
    return deg_kernel, agg_kernel


def _y_body(x_ref, w_ref, degp_ref, y_ref):
    deg = degp_ref[0] + degp_ref[1] + 1.0
    dis = lax.rsqrt(deg)
    xw = jnp.dot(x_ref[...], w_ref[...], preferred_element_type=jnp.float32)
    y_ref[...] = xw * dis


def _fin_body(aggp_ref, y_ref, degp_ref, wout_ref, bout_ref, out_ref, h_ref):
    deg = degp_ref[0] + degp_ref[1] + 1.0
    dis = lax.rsqrt(deg)
    s = aggp_ref[0] + aggp_ref[1] + y_ref[...]
    h = jnp.maximum(dis * s, 0.0)
    h_ref[...] = h
    out_ref[...] = (
        jnp.dot(h, wout_ref[...], preferred_element_type=jnp.float32)
        + bout_ref[...])


def kernel(x, edge_index, mask, W, W_out, b_out):
    del mask
    src = edge_index[0]
    dst = edge_index[1]
    pad = E_PAD - E
    src_p = jnp.concatenate(
        [src, jnp.zeros((pad,), jnp.int32)]).reshape(TOT_SEG, KSEG, G)
    dst_flat = jnp.concatenate([dst, jnp.full((pad,), N, jnp.int32)])
    dst_p = dst_flat.reshape(TOT_SEG, KSEG, G)
    eidx_p = jnp.stack([src_p, dst_p], axis=1)
    dst_p16 = dst_flat.reshape(NC, NS, NVEC, 16)

    zerosH = jnp.zeros((HR, 128), jnp.float32)
    zerosD = jnp.zeros((G, D), jnp.float32)
    iotaH = jnp.arange(HR, dtype=jnp.int32)

    deg_kernel, agg_kernel = _sc_kernels()
    degp = deg_kernel(dst_p16, zerosH, iotaH)
    degp3 = degp.reshape(NC, N_PAD, 1)

    y = pl.pallas_call(
        _y_body,
        grid=(GRID,),
        in_specs=[
            pl.BlockSpec((BN, D), lambda i: (i, 0)),
            pl.BlockSpec((D, D), lambda i: (0, 0)),
            pl.BlockSpec((NC, BN, 1), lambda i: (0, i, 0)),
        ],
        out_specs=pl.BlockSpec((BN, D), lambda i: (i, 0)),
        out_shape=jax.ShapeDtypeStruct((N, D), jnp.float32),
    )(x, W, degp3)

    aggp = agg_kernel(y, eidx_p, zerosD)

    out, h = pl.pallas_call(
        _fin_body,
        grid=(GRID,),
        in_specs=[
            pl.BlockSpec((NC, BN, D), lambda i: (0, i, 0)),
            pl.BlockSpec((BN, D), lambda i: (i, 0)),
            pl.BlockSpec((NC, BN, 1), lambda i: (0, i, 0)),
            pl.BlockSpec((D, T), lambda i: (0, 0)),
            pl.BlockSpec((1, T), lambda i: (0, 0)),
        ],
        out_specs=[
            pl.BlockSpec((BN, T), lambda i: (i, 0)),
            pl.BlockSpec((BN, D), lambda i: (i, 0)),
        ],
        out_shape=[
            jax.ShapeDtypeStruct((N, T), jnp.float32),
            jax.ShapeDtypeStruct((N, D), jnp.float32),
        ],
    )(aggp, y, degp3, W_out, b_out.reshape(1, T))

    return (out, h)


# speedup vs baseline: 1.0656x; 1.0656x over previous
import functools

import jax
import jax.numpy as jnp
from jax import lax
from jax.experimental import pallas as pl
from jax.experimental.pallas import tpu as pltpu
from jax.experimental.pallas import tpu_sc as plsc

N = 10000
E = 320000
D = 128
T = 2

NC = 2
NS = 16
LANES = 128
G = 80
KSEG = 8
NBUF = 4
SEG_C0 = 30
SEG_C1 = 2
TOT_SEG = NS * (SEG_C0 + SEG_C1)
EPT = 16 * KSEG * G
NVEC = EPT // 16
E_PAD = TOT_SEG * KSEG * G
N_PAD = 10240
ROWS_PER_SUB = N_PAD // NS
HR = N_PAD // 128
BN = 2000
GRID = N // BN


def _deg_body(dst_hbm, zeros_hbm, iota_hbm, deg_out,
              idx_v, hist, col_v, i80_v, acc):
    cid = lax.axis_index("c")
    sid = lax.axis_index("s")
    pltpu.sync_copy(zeros_hbm, hist)

    @pl.when(sid == 0)
    def _():
        pltpu.sync_copy(hist, acc)

    plsc.subcore_barrier()
    pltpu.sync_copy(dst_hbm.at[cid, sid], idx_v)
    pltpu.sync_copy(iota_hbm, i80_v)

    @pl.loop(0, NVEC)
    def _(j):
        d = idx_v[j]
        cnt, last = plsc.scan_count(d)
        r = lax.shift_right_logical(d, 7)
        c = lax.bitwise_and(d, 127)
        plsc.addupdate_scatter(hist, [r, c], cnt.astype(jnp.float32),
                               mask=last)

    pltpu.sync_copy(hist, acc.at[i80_v], add=True)
    plsc.subcore_barrier()
    pltpu.sync_copy(acc.at[pl.ds(sid * 5, 5)], hist.at[pl.ds(0, 5)])
    for kk in range(ROWS_PER_SUB // 16):
        col_v[pl.ds(kk * 16, 16)] = hist[kk // 8, pl.ds((kk % 8) * 16, 16)]
    pltpu.sync_copy(col_v, deg_out.at[cid, pl.ds(sid * ROWS_PER_SUB,
                                                 ROWS_PER_SUB)])


def _agg_body(y_hbm, eidx_hbm, zeros_hbm, agg_out,
              idx_v, bufs, gsems, ssems, acc):
    cid = lax.axis_index("c")
    sid = lax.axis_index("s")

    pltpu.sync_copy(zeros_hbm, bufs[0])
    for t in range(ROWS_PER_SUB // G):
        pltpu.sync_copy(
            bufs[0], acc.at[pl.ds(sid * ROWS_PER_SUB + t * G, G)])
    plsc.subcore_barrier()

    base = jnp.where(cid == 0, sid * SEG_C0, NS * SEG_C0 + sid * SEG_C1)
    nseg = jnp.where(cid == 0, SEG_C0, SEG_C1)

    @pl.loop(0, nseg)
    def _(seg):
        pltpu.sync_copy(eidx_hbm.at[base + seg], idx_v)
        hg = [None] * NBUF
        hs = [None] * NBUF

        def gather(j):
            b = j % NBUF
            if hs[b] is not None:
                hs[b].wait()
            hg[b] = pltpu.async_copy(
                y_hbm.at[idx_v.at[0, j]], bufs[b], gsems[b])

        def scat(j):
            b = j % NBUF
            hg[b].wait()
            hs[b] = pltpu.async_copy(
                bufs[b], acc.at[idx_v.at[1, j]], ssems[b], add=True)

        for j in range(KSEG):
            gather(j)
            jj = j - (NBUF - 1)
            if jj >= 0:
                scat(jj)
        for jj in range(KSEG - NBUF + 1, KSEG):
            scat(jj)
        for b in range(NBUF):
            hs[b].wait()

    plsc.subcore_barrier()
    for t in range(ROWS_PER_SUB // G):
        base = sid * ROWS_PER_SUB + t * G
        pltpu.sync_copy(acc.at[pl.ds(base, G)], bufs[0])
        pltpu.sync_copy(bufs[0], agg_out.at[cid, pl.ds(base, G)])


@functools.cache
def _sc_kernels():
    mesh = plsc.VectorSubcoreMesh(core_axis_name="c", subcore_axis_name="s")
    deg_kernel = pl.kernel(
        _deg_body,
        out_type=jax.ShapeDtypeStruct((NC, N_PAD), jnp.float32),
        mesh=mesh,
        compiler_params=pltpu.CompilerParams(needs_layout_passes=False),
        scratch_types=[
            pltpu.VMEM((NVEC, 16), jnp.int32),
            pltpu.VMEM((HR, 128), jnp.float32),
            pltpu.VMEM((ROWS_PER_SUB,), jnp.float32),
            pltpu.VMEM((HR,), jnp.int32),
            pltpu.VMEM_SHARED((HR, 128), jnp.float32),
        ],
    )
    agg_kernel = pl.kernel(
        _agg_body,
        out_type=jax.ShapeDtypeStruct((NC, N_PAD, D), jnp.float32),
        mesh=mesh,
        scratch_types=[
            pltpu.VMEM((2, KSEG, G), jnp.int32),
            [pltpu.VMEM((G, D), jnp.float32) for _ in range(NBUF)],
            [pltpu.SemaphoreType.DMA for _ in range(NBUF)],
            [pltpu.SemaphoreType.DMA for _ in range(NBUF)],
            pltpu.VMEM_SHARED((N_PAD, D), jnp.float32),
        ],
    )
    return deg_kernel, agg_kernel


def _y_body(x_ref, w_ref, degp_ref, y_ref):
    deg = degp_ref[0] + degp_ref[1] + 1.0
    dis = lax.rsqrt(deg)
    xw = jnp.dot(x_ref[...], w_ref[...], preferred_element_type=jnp.float32)
    y_ref[...] = xw * dis


def _fin_body(aggp_ref, y_ref, degp_ref, wout_ref, bout_ref, out_ref, h_ref):
    deg = degp_ref[0] + degp_ref[1] + 1.0
    dis = lax.rsqrt(deg)
    s = aggp_ref[0] + aggp_ref[1] + y_ref[...]
    h = jnp.maximum(dis * s, 0.0)
    h_ref[...] = h
    out_ref[...] = (
        jnp.dot(h, wout_ref[...], preferred_element_type=jnp.float32)
        + bout_ref[...])


def kernel(x, edge_index, mask, W, W_out, b_out):
    del mask
    src = edge_index[0]
    dst = edge_index[1]
    pad = E_PAD - E
    src_p = jnp.concatenate(
        [src, jnp.zeros((pad,), jnp.int32)]).reshape(TOT_SEG, KSEG, G)
    dst_flat = jnp.concatenate([dst, jnp.full((pad,), N, jnp.int32)])
    dst_p = dst_flat.reshape(TOT_SEG, KSEG, G)
    eidx_p = jnp.stack([src_p, dst_p], axis=1)
    dst_p16 = dst_flat.reshape(NC, NS, NVEC, 16)

    zerosH = jnp.zeros((HR, 128), jnp.float32)
    zerosD = jnp.zeros((G, D), jnp.float32)
    iotaH = jnp.arange(HR, dtype=jnp.int32)

    deg_kernel, agg_kernel = _sc_kernels()
    degp = deg_kernel(dst_p16, zerosH, iotaH)
    degp3 = degp.reshape(NC, N_PAD, 1)

    y = pl.pallas_call(
        _y_body,
        grid=(GRID,),
        in_specs=[
            pl.BlockSpec((BN, D), lambda i: (i, 0)),
            pl.BlockSpec((D, D), lambda i: (0, 0)),
            pl.BlockSpec((NC, BN, 1), lambda i: (0, i, 0)),
        ],
        out_specs=pl.BlockSpec((BN, D), lambda i: (i, 0)),
        out_shape=jax.ShapeDtypeStruct((N, D), jnp.float32),
    )(x, W, degp3)

    aggp = agg_kernel(y, eidx_p, zerosD)

    out, h = pl.pallas_call(
        _fin_body,
        grid=(GRID,),
        in_specs=[
            pl.BlockSpec((NC, BN, D), lambda i: (0, i, 0)),
            pl.BlockSpec((BN, D), lambda i: (i, 0)),
            pl.BlockSpec((NC, BN, 1), lambda i: (0, i, 0)),
            pl.BlockSpec((D, T), lambda i: (0, 0)),
            pl.BlockSpec((1, T), lambda i: (0, 0)),
        ],
        out_specs=[
            pl.BlockSpec((BN, T), lambda i: (i, 0)),
            pl.BlockSpec((BN, D), lambda i: (i, 0)),
        ],
        out_shape=[
            jax.ShapeDtypeStruct((N, T), jnp.float32),
            jax.ShapeDtypeStruct((N, D), jnp.float32),
        ],
    )(aggp, y, degp3, W_out, b_out.reshape(1, T))

    return (out, h)
